# baseline (speedup 1.0000x reference)
import functools

import jax
import jax.numpy as jnp
from jax import lax
from jax.experimental import pallas as pl
from jax.experimental.pallas import tpu as pltpu
from jax.experimental.pallas import tpu_sc as plsc

_D = 100
_DP = 112
_B = 4096 * 200
_NW = 32
_RW = _B // _NW
_IPG = 128
_G = 4
_CHUNK = _IPG * _G
_NCHUNK = _RW // _CHUNK

_mesh = plsc.VectorSubcoreMesh(core_axis_name="c", subcore_axis_name="s")


@functools.partial(
    pl.kernel,
    out_type=jax.ShapeDtypeStruct((_B * _D,), jnp.float32),
    mesh=_mesh,
    compiler_params=pltpu.CompilerParams(use_tc_tiling_on_sc=False),
    scratch_types=[
        pltpu.VMEM((_CHUNK,), jnp.int32),
        pltpu.VMEM((_CHUNK, _DP), jnp.float32),
        pltpu.VMEM((_CHUNK * _D + 16,), jnp.float32),
        pltpu.SemaphoreType.DMA,
    ],
)
def _emb_lookup(idx_hbm, table_hbm, out_hbm, idx_v, rows_v, comp_v, sem):
    wid = lax.axis_index("s") * 2 + lax.axis_index("c")
    base = wid * _RW

    def body(i, carry):
        off = base + i * _CHUNK
        pltpu.sync_copy(idx_hbm.at[pl.ds(off, _CHUNK)], idx_v)
        copies = [
            pltpu.async_copy(
                table_hbm.at[idx_v.at[pl.ds(j * _IPG, _IPG)]],
                rows_v.at[pl.ds(j * _IPG, _IPG)],
                sem,
            )
            for j in range(_G)
        ]
        for cp in copies:
            cp.wait()

        def row_body(r, c2):
            for k in range(7):
                comp_v[pl.ds(r * _D + k * 16, 16)] = rows_v[r, pl.ds(k * 16, 16)]
            return c2

        lax.fori_loop(0, _CHUNK, row_body, 0)
        pltpu.sync_copy(
            comp_v.at[pl.ds(0, _CHUNK * _D)],
            out_hbm.at[pl.ds(off * _D, _CHUNK * _D)],
        )
        return carry

    lax.fori_loop(0, _NCHUNK, body, 0)


def kernel(x, table):
    idx = x.reshape(-1).astype(jnp.int32)
    table_p = jnp.pad(table, ((0, 0), (0, _DP - _D)))
    out = _emb_lookup(idx, table_p)
    return out.reshape(x.shape + (_D,))


# speedup vs baseline: 2.2480x; 2.2480x over previous
import functools

import jax
import jax.numpy as jnp
from jax import lax
from jax.experimental import pallas as pl
from jax.experimental.pallas import tpu as pltpu
from jax.experimental.pallas import tpu_sc as plsc

_D = 100
_DP = 128
_B = 4096 * 200
_NW = 32
_RW = _B // _NW
_IPG = 128
_G = 2
_CHUNK = _IPG * _G
_NCHUNK = _RW // _CHUNK

_mesh = plsc.VectorSubcoreMesh(core_axis_name="c", subcore_axis_name="s")


@functools.partial(
    pl.kernel,
    out_type=jax.ShapeDtypeStruct((_B, _D), jnp.float32),
    mesh=_mesh,
    compiler_params=pltpu.CompilerParams(use_tc_tiling_on_sc=True),
    scratch_types=[
        pltpu.VMEM((_CHUNK,), jnp.int32),
        pltpu.VMEM((_CHUNK, _DP), jnp.float32),
        pltpu.VMEM((_CHUNK, _D), jnp.float32),
        pltpu.SemaphoreType.DMA,
    ],
)
def _emb_lookup(idx_hbm, table_hbm, out_hbm, idx_v, rows_v, comp_v, sem):
    wid = lax.axis_index("s") * 2 + lax.axis_index("c")
    base = wid * _RW

    def body(i, carry):
        off = base + i * _CHUNK
        pltpu.sync_copy(idx_hbm.at[pl.ds(off, _CHUNK)], idx_v)
        copies = [
            pltpu.async_copy(
                table_hbm.at[idx_v.at[pl.ds(j * _IPG, _IPG)]],
                rows_v.at[pl.ds(j * _IPG, _IPG)],
                sem,
            )
            for j in range(_G)
        ]
        for cp in copies:
            cp.wait()

        def row_body(r, c2):
            for o in (0, 16, 32, 48, 64, 80, 84):
                comp_v[r, pl.ds(o, 16)] = rows_v[r, pl.ds(o, 16)]
            return c2

        lax.fori_loop(0, _CHUNK, row_body, 0)
        pltpu.sync_copy(comp_v, out_hbm.at[pl.ds(off, _CHUNK)])
        return carry

    lax.fori_loop(0, _NCHUNK, body, 0)


def kernel(x, table):
    idx = x.reshape(-1).astype(jnp.int32)
    table_p = jnp.pad(table, ((0, 0), (0, _DP - _D)))
    out = _emb_lookup(idx, table_p)
    return out.reshape(x.shape + (_D,))


# speedup vs baseline: 2.7505x; 1.2235x over previous
import functools

import jax
import jax.numpy as jnp
from jax import lax
from jax.experimental import pallas as pl
from jax.experimental.pallas import tpu as pltpu
from jax.experimental.pallas import tpu_sc as plsc

_D = 100
_DP = 128
_T = 200
_NB = 4096
_B = _NB * _T
_NW = 32
_WB = _NB // _NW
_HB = _WB // 2

_mesh = plsc.VectorSubcoreMesh(core_axis_name="c", subcore_axis_name="s")


def _pad_body(t_ref, o_ref):
    o_ref[:, :_D] = t_ref[...]
    o_ref[:, _D:] = jnp.zeros((t_ref.shape[0], _DP - _D), jnp.float32)


def _pad_table(t):
    rows, blk = t.shape[0], 2000
    return pl.pallas_call(
        _pad_body,
        grid=(rows // blk,),
        in_specs=[pl.BlockSpec((blk, _D), lambda i: (i, 0))],
        out_specs=pl.BlockSpec((blk, _DP), lambda i: (i, 0)),
        out_shape=jax.ShapeDtypeStruct((rows, _DP), jnp.float32),
    )(t)


@functools.partial(
    pl.kernel,
    out_type=jax.ShapeDtypeStruct((_B, _D), jnp.float32),
    mesh=_mesh,
    compiler_params=pltpu.CompilerParams(use_tc_tiling_on_sc=True),
    scratch_types=[
        pltpu.VMEM((_HB, _T), jnp.int32),
        pltpu.VMEM((2 * _T, _DP), jnp.float32),
        pltpu.VMEM((2 * _T, _D), jnp.float32),
        pltpu.SemaphoreType.DMA,
    ],
)
def _emb_lookup(x_hbm, table_hbm, out_hbm, idx_v, rows_v, comp_v, sem):
    wid = lax.axis_index("s") * 2 + lax.axis_index("c")
    wb = wid * _WB

    for h in range(2):
        pltpu.sync_copy(x_hbm.at[pl.ds(wb + h * _HB, _HB)], idx_v)
        out_base = (wb + h * _HB) * _T

        def body(it, carry):
            copies = []
            for j in range(2):
                r = 2 * it + j
                copies.append(pltpu.async_copy(
                    table_hbm.at[idx_v.at[r, pl.ds(0, 128)]],
                    rows_v.at[pl.ds(_T * j, 128)], sem))
                copies.append(pltpu.async_copy(
                    table_hbm.at[idx_v.at[r, pl.ds(128, _T - 128)]],
                    rows_v.at[pl.ds(_T * j + 128, _T - 128)], sem))
            for cp in copies:
                cp.wait()

            def row_body(r2, c2):
                for o in (0, 16, 32, 48, 64, 80, 84):
                    comp_v[r2, pl.ds(o, 16)] = rows_v[r2, pl.ds(o, 16)]
                return c2

            lax.fori_loop(0, 2 * _T, row_body, 0)
            pltpu.sync_copy(
                comp_v, out_hbm.at[pl.ds(out_base + it * 2 * _T, 2 * _T)])
            return carry

        lax.fori_loop(0, _HB // 2, body, 0)


def kernel(x, table):
    table_p = _pad_table(table)
    out = _emb_lookup(x.astype(jnp.int32), table_p)
    return out.reshape(x.shape + (_D,))
